# baseline (speedup 1.0000x reference)
import functools

import jax
import jax.numpy as jnp
from jax import lax
from jax.experimental import pallas as pl
from jax.experimental.pallas import tpu as pltpu
from jax.experimental.pallas import tpu_sc as plsc

D_MODEL = 128
SCALE = float(D_MODEL) ** 0.5

_info = plsc.get_sparse_core_info()
_NC, _NS = _info.num_cores, _info.num_subcores
_NW = _NC * _NS

_B = 4096 * 200
_CHUNK = 128
_ROWS_PER_W = _B // _NW
_CHUNKS_PER_W = _ROWS_PER_W // _CHUNK
_SUB = 2
_BIG = _SUB * _CHUNK
_NBIG = _ROWS_PER_W // _BIG


@functools.partial(
    pl.kernel,
    mesh=plsc.VectorSubcoreMesh(core_axis_name="c", subcore_axis_name="s"),
    out_type=jax.ShapeDtypeStruct((_B, D_MODEL), jnp.float32),
    scratch_types=(
        [pltpu.VMEM((_CHUNKS_PER_W, _CHUNK), jnp.int32)]
        + [pltpu.VMEM((_BIG, D_MODEL), jnp.float32) for _ in range(3)]
        + [pltpu.SemaphoreType.DMA for _ in range(6)]
    ),
)
def _sc_gather(table_hbm, idx_hbm, out_hbm, idx_v, b0, b1, b2,
               g0, g1, g2, s0, s1, s2):
    bufs = [b0, b1, b2]
    gsems = [g0, g1, g2]
    ssems = [s0, s1, s2]
    wid = lax.axis_index("s") * _NC + lax.axis_index("c")
    ibase = wid * _CHUNKS_PER_W
    obase = wid * _ROWS_PER_W
    pltpu.sync_copy(idx_hbm.at[pl.ds(ibase, _CHUNKS_PER_W)], idx_v)

    def gather(m, k):
        for h in range(_SUB):
            pltpu.make_async_copy(
                table_hbm.at[idx_v.at[_SUB * m + h]],
                bufs[k].at[pl.ds(h * _CHUNK, _CHUNK)],
                gsems[k],
            ).start()

    def gwait(k):
        for h in range(_SUB):
            pltpu.make_async_copy(
                table_hbm.at[idx_v.at[0]],
                bufs[k].at[pl.ds(h * _CHUNK, _CHUNK)],
                gsems[k],
            ).wait()

    def scatter(m, k):
        pltpu.make_async_copy(
            bufs[k], out_hbm.at[pl.ds(obase + m * _BIG, _BIG)], ssems[k]
        ).start()

    def swait(k):
        pltpu.make_async_copy(
            bufs[k], out_hbm.at[pl.ds(obase, _BIG)], ssems[k]
        ).wait()

    def scale_buf(k):
        buf = bufs[k]

        def row_body(i, carry):
            for r_off in range(4):
                r = 4 * i + r_off
                for c in range(D_MODEL // 16):
                    sl = pl.ds(c * 16, 16)
                    buf[r, sl] = buf[r, sl] * SCALE
            return carry

        lax.fori_loop(0, _BIG // 4, row_body, 0)

    def step(j, s, sprev):
        gwait(sprev)
        scale_buf(sprev)
        scatter(j - 1, sprev)
        swait(s)
        gather(j, s)

    gather(0, 0)
    for j in (1, 2):
        gwait(j - 1)
        scale_buf(j - 1)
        scatter(j - 1, j - 1)
        gather(j, j)
    gwait(2)
    scale_buf(2)
    scatter(2, 2)
    swait(0)
    gather(3, 0)

    def body(i, carry):
        j0 = 4 + 3 * i
        for t in range(3):
            j = j0 + t
            step(j, (4 + t) % 3, (3 + t) % 3)
        return carry

    lax.fori_loop(0, (_NBIG - 4) // 3, body, 0)

    gwait((_NBIG - 1) % 3)
    scale_buf((_NBIG - 1) % 3)
    scatter(_NBIG - 1, (_NBIG - 1) % 3)
    for k in range(3):
        swait(k)


def kernel(x, table):
    idx = x.reshape(_B // _CHUNK, _CHUNK).astype(jnp.int32)
    out = _sc_gather(table, idx)
    return out.reshape(x.shape[0], x.shape[1], D_MODEL)


# speedup vs baseline: 1.2651x; 1.2651x over previous
import functools

import jax
import jax.numpy as jnp
from jax import lax
from jax.experimental import pallas as pl
from jax.experimental.pallas import tpu as pltpu
from jax.experimental.pallas import tpu_sc as plsc

D_MODEL = 128
SCALE = float(D_MODEL) ** 0.5

_info = plsc.get_sparse_core_info()
_NC, _NS = _info.num_cores, _info.num_subcores
_NW = _NC * _NS

_B = 4096 * 200
_CHUNK = 128
_ROWS_PER_W = _B // _NW
_CHUNKS_PER_W = _ROWS_PER_W // _CHUNK


@functools.partial(
    pl.kernel,
    mesh=plsc.VectorSubcoreMesh(core_axis_name="c", subcore_axis_name="s"),
    out_type=jax.ShapeDtypeStruct((_B, D_MODEL), jnp.float32),
    scratch_types=(
        [pltpu.VMEM((_CHUNKS_PER_W, _CHUNK), jnp.int32)]
        + [pltpu.VMEM((_CHUNK, D_MODEL), jnp.float32) for _ in range(6)]
        + [pltpu.SemaphoreType.DMA for _ in range(12)]
    ),
)
def _sc_gather(table_hbm, idx_hbm, out_hbm, idx_v, b0, b1, b2, b3, b4, b5,
               g0, g1, g2, g3, g4, g5, s0, s1, s2, s3, s4, s5):
    bufs = [b0, b1, b2, b3, b4, b5]
    gsems = [g0, g1, g2, g3, g4, g5]
    ssems = [s0, s1, s2, s3, s4, s5]
    wid = lax.axis_index("s") * _NC + lax.axis_index("c")
    ibase = wid * _CHUNKS_PER_W
    obase = wid * _ROWS_PER_W
    pltpu.sync_copy(idx_hbm.at[pl.ds(ibase, _CHUNKS_PER_W)], idx_v)

    def gather(j, k):
        pltpu.make_async_copy(table_hbm.at[idx_v.at[j]], bufs[k], gsems[k]).start()

    def gwait(k):
        pltpu.make_async_copy(table_hbm.at[idx_v.at[0]], bufs[k], gsems[k]).wait()

    def scatter(j, k):
        pltpu.make_async_copy(
            bufs[k], out_hbm.at[pl.ds(obase + j * _CHUNK, _CHUNK)], ssems[k]
        ).start()

    def swait(k):
        pltpu.make_async_copy(
            bufs[k], out_hbm.at[pl.ds(obase, _CHUNK)], ssems[k]
        ).wait()

    def scale_buf(k):
        buf = bufs[k]

        def row_body(i, carry):
            for r_off in range(4):
                r = 4 * i + r_off
                for c in range(D_MODEL // 16):
                    sl = pl.ds(c * 16, 16)
                    buf[r, sl] = buf[r, sl] * SCALE
            return carry

        lax.fori_loop(0, _CHUNK // 4, row_body, 0)

    for j in range(6):
        gather(j, j)
        if j >= 2:
            gwait(j - 2)
            scale_buf(j - 2)
            scatter(j - 2, j - 2)

    def step(j, k):
        swait(k)
        gather(j, k)
        gwait((k + 4) % 6)
        scale_buf((k + 4) % 6)
        scatter(j - 2, (k + 4) % 6)

    def body(i, carry):
        j0 = 6 + 6 * i
        for t in range(6):
            step(j0 + t, t)
        return carry

    lax.fori_loop(0, (_CHUNKS_PER_W - 8) // 6, body, 0)

    for j in (_CHUNKS_PER_W - 2, _CHUNKS_PER_W - 1):
        step(j, j % 6)
    for j in (_CHUNKS_PER_W - 2, _CHUNKS_PER_W - 1):
        gwait(j % 6)
        scale_buf(j % 6)
        scatter(j, j % 6)
    for k in range(6):
        swait(k)


def kernel(x, table):
    idx = x.reshape(_B // _CHUNK, _CHUNK).astype(jnp.int32)
    out = _sc_gather(table, idx)
    return out.reshape(x.shape[0], x.shape[1], D_MODEL)
